# Initial kernel scaffold; baseline (speedup 1.0000x reference)
import jax
import jax.numpy as jnp
from jax.experimental import pallas as pl


def kernel(x, edge_index, edge_weight, state_of_county, W1, b1, W2, b2):
    raise NotImplementedError("write your pallas kernel here")


# speedup vs baseline: 16.2063x; 16.2063x over previous
import dataclasses
import functools

import jax
import jax.numpy as jnp
from jax import lax
from jax.experimental import pallas as pl
from jax.experimental.pallas import tpu as pltpu
from jax.experimental.pallas import tpu_sc as plsc

N = 10000
E = 320000
D = 128
S = 50

NC = 2
NS = 16
NW = NC * NS
EPW = E // NW
CH = 80
NCH = EPW // CH
NBLK = 5
BCH = NCH // NBLK
EB = EPW // NBLK
RPT = 632
RPT_LAST = N - RPT * (NS - 1)
NODE_CH = N // CH
SP = 56

_mesh = plsc.VectorSubcoreMesh(core_axis_name="c", subcore_axis_name="s")
_sc_params = pltpu.CompilerParams()
if "needs_layout_passes" in pltpu.CompilerParams.__dataclass_fields__:
    _sc_params = dataclasses.replace(_sc_params, needs_layout_passes=False)


@functools.partial(
    pl.kernel,
    mesh=_mesh,
    compiler_params=_sc_params,
    out_type=jax.ShapeDtypeStruct((NW, N), jnp.float32),
    scratch_types=[
        pltpu.VMEM((N,), jnp.float32),
        pltpu.VMEM((EB,), jnp.int32),
        pltpu.VMEM((EB,), jnp.float32),
    ],
)
def _sc_degree(dst_hbm, ew_hbm, deg_out, deg_v, dst_v, w_v):
    c = lax.axis_index("c")
    s = lax.axis_index("s")
    wid = c * NS + s

    @pl.loop(0, N // 16)
    def _zero(i):
        deg_v[pl.ds(i * 16, 16)] = jnp.zeros((16,), jnp.float32)

    @pl.loop(0, NBLK)
    def _block(b):
        base = wid * EPW + b * EB
        pltpu.sync_copy(dst_hbm.at[pl.ds(base, EB)], dst_v)
        pltpu.sync_copy(ew_hbm.at[pl.ds(base, EB)], w_v)

        @pl.loop(0, EB // 16)
        def _acc(i):
            idx = dst_v[pl.ds(i * 16, 16)]
            w16 = w_v[pl.ds(i * 16, 16)]
            plsc.addupdate_scatter(deg_v, [idx], w16)

    pltpu.sync_copy(deg_v, deg_out.at[wid])


@functools.partial(
    pl.kernel,
    mesh=_mesh,
    compiler_params=_sc_params,
    out_type=jax.ShapeDtypeStruct((NC, N, D), jnp.float32),
    scratch_types=[
        pltpu.VMEM_SHARED((N, D), jnp.float32),
        pltpu.VMEM((BCH, CH), jnp.int32),
        pltpu.VMEM((BCH, CH), jnp.int32),
        pltpu.VMEM((BCH, CH), jnp.float32),
        pltpu.VMEM((CH, D), jnp.float32),
        pltpu.SemaphoreType.DMA,
    ],
)
def _sc_aggregate(hs_hbm, src_r_hbm, dst_r_hbm, ew_r_hbm, zeros_hbm, agg_out,
                  acc, src_v, dst_v, w_v, rows, sem):
    c = lax.axis_index("c")
    s = lax.axis_index("s")
    wid = c * NS + s

    @pl.when(s < NS - 1)
    def _zero_main():
        pltpu.sync_copy(zeros_hbm.at[pl.ds(s * RPT, RPT)],
                        acc.at[pl.ds(s * RPT, RPT)])

    @pl.when(s == NS - 1)
    def _zero_last():
        pltpu.sync_copy(zeros_hbm.at[pl.ds((NS - 1) * RPT, RPT_LAST)],
                        acc.at[pl.ds((NS - 1) * RPT, RPT_LAST)])

    plsc.subcore_barrier()

    @pl.loop(0, NBLK)
    def _block(b):
        pltpu.sync_copy(src_r_hbm.at[wid, b], src_v)
        pltpu.sync_copy(dst_r_hbm.at[wid, b], dst_v)
        pltpu.sync_copy(ew_r_hbm.at[wid, b], w_v)

        @pl.loop(0, BCH)
        def _chunk(k):
            pltpu.async_copy(hs_hbm.at[src_v.at[k]], rows, sem).wait()

            @pl.loop(0, CH // 16)
            def _scale(jb):
                w16 = w_v[k, pl.ds(jb * 16, 16)]
                for l in range(16):
                    wj = w16[l]
                    j = jb * 16 + l
                    for t in range(D // 16):
                        sl = pl.ds(t * 16, 16)
                        rows[j, sl] = rows[j, sl] * wj

            pltpu.sync_copy(rows, acc.at[dst_v.at[k]], add=True)

    plsc.subcore_barrier()

    @pl.when(s < NS - 1)
    def _flush_main():
        pltpu.sync_copy(acc.at[pl.ds(s * RPT, RPT)],
                        agg_out.at[c, pl.ds(s * RPT, RPT)])

    @pl.when(s == NS - 1)
    def _flush_last():
        pltpu.sync_copy(acc.at[pl.ds((NS - 1) * RPT, RPT_LAST)],
                        agg_out.at[c, pl.ds((NS - 1) * RPT, RPT_LAST)])


@functools.partial(
    pl.kernel,
    mesh=_mesh,
    compiler_params=_sc_params,
    out_type=(jax.ShapeDtypeStruct((NC, SP, D), jnp.float32),
              jax.ShapeDtypeStruct((NC, NS, 64), jnp.float32)),
    scratch_types=[
        pltpu.VMEM_SHARED((SP, D), jnp.float32),
        pltpu.VMEM((1, CH), jnp.int32),
        pltpu.VMEM((CH, D), jnp.float32),
        pltpu.VMEM((64,), jnp.float32),
    ],
)
def _sc_segment(h_hbm, st_hbm, zeros_hbm, seg_out, cnt_out,
                acc, sidx, rows, cnt_v):
    c = lax.axis_index("c")
    s = lax.axis_index("s")
    wid = c * NS + s

    @pl.when(s < SP // 8)
    def _zero_acc():
        pltpu.sync_copy(zeros_hbm.at[pl.ds(s * 8, 8)], acc.at[pl.ds(s * 8, 8)])

    for t in range(4):
        cnt_v[pl.ds(t * 16, 16)] = jnp.zeros((16,), jnp.float32)
    plsc.subcore_barrier()

    @pl.loop(wid, NODE_CH, step=NW)
    def _chunk(k):
        base = k * CH
        pltpu.sync_copy(h_hbm.at[pl.ds(base, CH)], rows)
        pltpu.sync_copy(st_hbm.at[pl.ds(base, CH)], sidx.at[0])
        pltpu.sync_copy(rows, acc.at[sidx.at[0]], add=True)
        for t in range(CH // 16):
            st16 = sidx[0, pl.ds(t * 16, 16)]
            plsc.addupdate_scatter(cnt_v, [st16], jnp.ones((16,), jnp.float32))

    pltpu.sync_copy(cnt_v, cnt_out.at[c, s])
    plsc.subcore_barrier()

    @pl.when(s < SP // 8)
    def _flush():
        pltpu.sync_copy(acc.at[pl.ds(s * 8, 8)], seg_out.at[c, pl.ds(s * 8, 8)])


def _tc1_body(x_ref, w1_ref, degp_ref, hs1_ref, dinv_ref):
    deg = jnp.sum(degp_ref[...], axis=0) + 1.0
    dinv = jnp.where(deg > 0, lax.rsqrt(deg), 0.0)
    dinv_ref[...] = dinv[:, None]
    h1 = jnp.dot(x_ref[...], w1_ref[...], preferred_element_type=jnp.float32)
    hs1_ref[...] = h1 * dinv[:, None]


def _tc2_body(agg_ref, hs1_ref, dinv_ref, b1_ref, w2_ref, hs2_ref):
    dinv = dinv_ref[...]
    pre = (agg_ref[0] + agg_ref[1] + hs1_ref[...]) * dinv + b1_ref[...][None, :]
    a1 = jnp.maximum(pre, 0.0)
    h2 = jnp.dot(a1, w2_ref[...], preferred_element_type=jnp.float32)
    hs2_ref[...] = h2 * dinv


def _tc3_body(agg_ref, hs2_ref, dinv_ref, b2_ref, h_ref):
    h_ref[...] = ((agg_ref[0] + agg_ref[1] + hs2_ref[...]) * dinv_ref[...]
                  + b2_ref[...][None, :])


def _tc4_body(seg_ref, cnt_ref, out_ref):
    cnt = jnp.sum(cnt_ref[...], axis=(0, 1))[:S]
    total = seg_ref[0, :S] + seg_ref[1, :S]
    out_ref[...] = total / jnp.maximum(cnt, 1.0)[:, None]


def kernel(x, edge_index, edge_weight, state_of_county, W1, b1, W2, b2):
    src_r = edge_index[0].reshape(NW, NBLK, BCH, CH)
    dst_r = edge_index[1].reshape(NW, NBLK, BCH, CH)
    ew_r = edge_weight.reshape(NW, NBLK, BCH, CH)
    zeros_nd = jnp.zeros((N, D), jnp.float32)

    deg_parts = _sc_degree(edge_index[1], edge_weight)

    hs1, dinv = pl.pallas_call(
        _tc1_body,
        out_shape=(jax.ShapeDtypeStruct((N, D), jnp.float32),
                   jax.ShapeDtypeStruct((N, 1), jnp.float32)),
    )(x, W1, deg_parts)

    agg1 = _sc_aggregate(hs1, src_r, dst_r, ew_r, zeros_nd)

    hs2 = pl.pallas_call(
        _tc2_body,
        out_shape=jax.ShapeDtypeStruct((N, D), jnp.float32),
    )(agg1, hs1, dinv, b1, W2)

    agg2 = _sc_aggregate(hs2, src_r, dst_r, ew_r, zeros_nd)

    h_final = pl.pallas_call(
        _tc3_body,
        out_shape=jax.ShapeDtypeStruct((N, D), jnp.float32),
    )(agg2, hs2, dinv, b2)

    seg, cnt = _sc_segment(h_final, state_of_county, zeros_nd)

    out = pl.pallas_call(
        _tc4_body,
        out_shape=jax.ShapeDtypeStruct((S, D), jnp.float32),
    )(seg, cnt)
    return out
